# baseline (speedup 1.0000x reference)
import jax
import jax.numpy as jnp
from jax.experimental import pallas as pl
from jax.experimental.pallas import tpu as pltpu

_TM = 200


def _support(f_ref, w_ref, s_ref):
    @pl.when(pl.program_id(0) == 0)
    def _():
        s = jnp.dot(f_ref[...], w_ref[...], preferred_element_type=jnp.float32)
        s_ref[...] = jnp.tanh(s).astype(jnp.bfloat16)


def _layer_cast_body(f_ref, w_ref, adj_ref, out_ref, adj_bf_ref, s_ref):
    _support(f_ref, w_ref, s_ref)
    a = adj_ref[...].astype(jnp.bfloat16)
    adj_bf_ref[...] = a
    out_ref[...] = jnp.dot(a, s_ref[...], preferred_element_type=jnp.float32)


def _layer_body(f_ref, w_ref, adj_ref, out_ref, s_ref):
    _support(f_ref, w_ref, s_ref)
    out_ref[...] = jnp.dot(adj_ref[...], s_ref[...],
                           preferred_element_type=jnp.float32)


def _gcn_layer(features, W, adj, tm, emit_bf16_adj=False):
    N, d_in = features.shape
    d_out = W.shape[1]
    out_shape = jax.ShapeDtypeStruct((N, d_out), jnp.float32)
    out_spec = pl.BlockSpec((tm, d_out), lambda i: (i, 0))
    if emit_bf16_adj:
        body = _layer_cast_body
        out_shape = [out_shape, jax.ShapeDtypeStruct((N, N), jnp.bfloat16)]
        out_spec = [out_spec, pl.BlockSpec((tm, N), lambda i: (i, 0))]
    else:
        body = _layer_body
    return pl.pallas_call(
        body,
        grid=(N // tm,),
        in_specs=[
            pl.BlockSpec((N, d_in), lambda i: (0, 0)),
            pl.BlockSpec((d_in, d_out), lambda i: (0, 0)),
            pl.BlockSpec((tm, N), lambda i: (i, 0)),
        ],
        out_specs=out_spec,
        out_shape=out_shape,
        scratch_shapes=[pltpu.VMEM((N, d_out), jnp.bfloat16)],
    )(features, W, adj)


def _final_body(zh_blk_ref, zh_full_ref, out_ref, zt_ref):
    @pl.when(pl.program_id(0) == 0)
    def _():
        zt_ref[...] = zh_full_ref[...].T.astype(jnp.bfloat16)

    lhs = zh_blk_ref[...].astype(jnp.bfloat16)
    acc = jnp.dot(lhs, zt_ref[...], preferred_element_type=jnp.float32)
    out_ref[...] = 0.5 * jnp.tanh(0.5 * acc) + 0.5


def _gram_sigmoid(z_hat, tm):
    N, d = z_hat.shape
    return pl.pallas_call(
        _final_body,
        grid=(N // tm,),
        in_specs=[
            pl.BlockSpec((tm, d), lambda i: (i, 0)),
            pl.BlockSpec((N, d), lambda i: (0, 0)),
        ],
        out_specs=pl.BlockSpec((tm, N), lambda i: (i, 0)),
        out_shape=jax.ShapeDtypeStruct((N, N), jnp.float32),
        scratch_shapes=[pltpu.VMEM((d, N), jnp.bfloat16)],
    )(z_hat, z_hat)


def kernel(z_igae, adj, W4, W5, W6):
    N = adj.shape[0]
    tm = _TM if N % _TM == 0 else N
    z1, adj_bf = _gcn_layer(z_igae, W4, adj, tm, emit_bf16_adj=True)
    z2 = _gcn_layer(z1, W5, adj_bf, tm)
    z_hat = _gcn_layer(z2, W6, adj_bf, tm)
    z_hat_adj = _gram_sigmoid(z_hat, tm)
    return (z_hat, z_hat_adj)


# speedup vs baseline: 1.0678x; 1.0678x over previous
import jax
import jax.numpy as jnp
from jax.experimental import pallas as pl
from jax.experimental.pallas import tpu as pltpu

_TM = 200


def _support(f_ref, w_ref, s_ref):
    @pl.when(pl.program_id(0) == 0)
    def _():
        s = jnp.dot(f_ref[...], w_ref[...], preferred_element_type=jnp.float32)
        s_ref[...] = jnp.tanh(s).astype(jnp.bfloat16)


def _layer_cast_body(f_ref, w_ref, adj_ref, out_ref, adj_bf_ref, s_ref):
    _support(f_ref, w_ref, s_ref)
    a = adj_ref[...].astype(jnp.bfloat16)
    adj_bf_ref[...] = a
    out_ref[...] = jnp.dot(a, s_ref[...], preferred_element_type=jnp.float32)


def _layer_body(f_ref, w_ref, adj_ref, out_ref, s_ref):
    _support(f_ref, w_ref, s_ref)
    out_ref[...] = jnp.dot(adj_ref[...], s_ref[...],
                           preferred_element_type=jnp.float32)


def _gcn_layer(features, W, adj, tm, emit_bf16_adj=False):
    N, d_in = features.shape
    d_out = W.shape[1]
    out_shape = jax.ShapeDtypeStruct((N, d_out), jnp.float32)
    out_spec = pl.BlockSpec((tm, d_out), lambda i: (i, 0))
    if emit_bf16_adj:
        body = _layer_cast_body
        out_shape = [out_shape, jax.ShapeDtypeStruct((N, N), jnp.bfloat16)]
        out_spec = [out_spec, pl.BlockSpec((tm, N), lambda i: (i, 0))]
    else:
        body = _layer_body
    return pl.pallas_call(
        body,
        grid=(N // tm,),
        in_specs=[
            pl.BlockSpec((N, d_in), lambda i: (0, 0)),
            pl.BlockSpec((d_in, d_out), lambda i: (0, 0)),
            pl.BlockSpec((tm, N), lambda i: (i, 0)),
        ],
        out_specs=out_spec,
        out_shape=out_shape,
        scratch_shapes=[pltpu.VMEM((N, d_out), jnp.bfloat16)],
    )(features, W, adj)


def _final_body(zh_blk_ref, zh_full_ref, out_ref, zt_ref):
    @pl.when(pl.program_id(0) == 0)
    def _():
        zt_ref[...] = zh_full_ref[...].T.astype(jnp.bfloat16)

    lhs = zh_blk_ref[...].astype(jnp.bfloat16)
    acc = jnp.dot(lhs, zt_ref[...], preferred_element_type=jnp.float32)
    out_ref[...] = 0.5 * jnp.tanh(0.5 * acc) + 0.5


def _gram_sigmoid(z_hat, tm):
    N, d = z_hat.shape
    return pl.pallas_call(
        _final_body,
        grid=(N // tm,),
        in_specs=[
            pl.BlockSpec((tm, d), lambda i: (i, 0)),
            pl.BlockSpec((N, d), lambda i: (0, 0)),
        ],
        out_specs=pl.BlockSpec((tm, N), lambda i: (i, 0)),
        out_shape=jax.ShapeDtypeStruct((N, N), jnp.float32),
        scratch_shapes=[pltpu.VMEM((d, N), jnp.bfloat16)],
    )(z_hat, z_hat)


def kernel(z_igae, adj, W4, W5, W6):
    N = adj.shape[0]
    tm = _TM if N % _TM == 0 else N
    tm2 = 2 * _TM if N % (2 * _TM) == 0 else tm
    z1, adj_bf = _gcn_layer(z_igae, W4, adj, tm, emit_bf16_adj=True)
    z2 = _gcn_layer(z1, W5, adj_bf, tm2)
    z_hat = _gcn_layer(z2, W6, adj_bf, tm2)
    z_hat_adj = _gram_sigmoid(z_hat, tm2)
    return (z_hat, z_hat_adj)


# speedup vs baseline: 1.3158x; 1.2323x over previous
import jax
import jax.numpy as jnp
from jax.experimental import pallas as pl
from jax.experimental.pallas import tpu as pltpu

_TM = 200


_F8 = jnp.float8_e4m3fn


def _support(f_ref, w_ref, s_ref):
    @pl.when(pl.program_id(0) == 0)
    def _():
        s = jnp.dot(f_ref[...], w_ref[...], preferred_element_type=jnp.float32)
        s_ref[...] = jnp.tanh(s).astype(s_ref.dtype)


def _layer_cast_body(f_ref, w_ref, adj_ref, out_ref, adj_f8_ref, s_ref):
    _support(f_ref, w_ref, s_ref)
    a32 = adj_ref[...]
    adj_f8_ref[...] = a32.astype(_F8)
    out_ref[...] = jnp.dot(a32.astype(jnp.bfloat16), s_ref[...],
                           preferred_element_type=jnp.float32)


def _layer_body(f_ref, w_ref, adj_ref, out_ref, s_ref):
    _support(f_ref, w_ref, s_ref)
    out_ref[...] = jnp.dot(adj_ref[...], s_ref[...],
                           preferred_element_type=jnp.float32)


def _gcn_layer(features, W, adj, tm, emit_f8_adj=False):
    N, d_in = features.shape
    d_out = W.shape[1]
    out_shape = jax.ShapeDtypeStruct((N, d_out), jnp.float32)
    out_spec = pl.BlockSpec((tm, d_out), lambda i: (i, 0))
    if emit_f8_adj:
        body = _layer_cast_body
        s_dtype = jnp.bfloat16
        out_shape = [out_shape, jax.ShapeDtypeStruct((N, N), _F8)]
        out_spec = [out_spec, pl.BlockSpec((tm, N), lambda i: (i, 0))]
    else:
        body = _layer_body
        s_dtype = adj.dtype
    return pl.pallas_call(
        body,
        grid=(N // tm,),
        in_specs=[
            pl.BlockSpec((N, d_in), lambda i: (0, 0)),
            pl.BlockSpec((d_in, d_out), lambda i: (0, 0)),
            pl.BlockSpec((tm, N), lambda i: (i, 0)),
        ],
        out_specs=out_spec,
        out_shape=out_shape,
        scratch_shapes=[pltpu.VMEM((N, d_out), s_dtype)],
    )(features, W, adj)


def _final_body(zh_blk_ref, zh_full_ref, out_ref, zt_ref):
    @pl.when(pl.program_id(0) == 0)
    def _():
        zt_ref[...] = zh_full_ref[...].T.astype(jnp.bfloat16)

    lhs = zh_blk_ref[...].astype(jnp.bfloat16)
    acc = jnp.dot(lhs, zt_ref[...], preferred_element_type=jnp.float32)
    out_ref[...] = 0.5 * jnp.tanh(0.5 * acc) + 0.5


def _gram_sigmoid(z_hat, tm):
    N, d = z_hat.shape
    return pl.pallas_call(
        _final_body,
        grid=(N // tm,),
        in_specs=[
            pl.BlockSpec((tm, d), lambda i: (i, 0)),
            pl.BlockSpec((N, d), lambda i: (0, 0)),
        ],
        out_specs=pl.BlockSpec((tm, N), lambda i: (i, 0)),
        out_shape=jax.ShapeDtypeStruct((N, N), jnp.float32),
        scratch_shapes=[pltpu.VMEM((d, N), jnp.bfloat16)],
    )(z_hat, z_hat)


def kernel(z_igae, adj, W4, W5, W6):
    N = adj.shape[0]
    tm = _TM if N % _TM == 0 else N
    tm2 = 2 * _TM if N % (2 * _TM) == 0 else tm
    z1, adj_f8 = _gcn_layer(z_igae, W4, adj, tm, emit_f8_adj=True)
    z2 = _gcn_layer(z1, W5, adj_f8, tm2)
    z_hat = _gcn_layer(z2, W6, adj_f8, tm2)
    z_hat_adj = _gram_sigmoid(z_hat, tm2)
    return (z_hat, z_hat_adj)


# speedup vs baseline: 1.3746x; 1.0447x over previous
import jax
import jax.numpy as jnp
from jax.experimental import pallas as pl
from jax.experimental.pallas import tpu as pltpu

_TM = 200


_F8 = jnp.float8_e4m3fn


def _support(f_ref, w_ref, s_ref):
    @pl.when(pl.program_id(0) == 0)
    def _():
        s = jnp.dot(f_ref[...].astype(jnp.bfloat16),
                    w_ref[...].astype(jnp.bfloat16),
                    preferred_element_type=jnp.float32)
        s_ref[...] = jnp.tanh(s).astype(s_ref.dtype)


def _layer_cast_body(f_ref, w_ref, adj_ref, out_ref, adj_f8_ref, s_ref):
    _support(f_ref, w_ref, s_ref)
    a32 = adj_ref[...]
    adj_f8_ref[...] = a32.astype(_F8)
    out_ref[...] = jnp.dot(a32.astype(jnp.bfloat16), s_ref[...],
                           preferred_element_type=jnp.float32)


def _layer_body(f_ref, w_ref, adj_ref, out_ref, s_ref):
    _support(f_ref, w_ref, s_ref)
    out_ref[...] = jnp.dot(adj_ref[...], s_ref[...],
                           preferred_element_type=jnp.float32)


def _gcn_layer(features, W, adj, tm, emit_f8_adj=False):
    N, d_in = features.shape
    d_out = W.shape[1]
    out_shape = jax.ShapeDtypeStruct((N, d_out), jnp.float32)
    out_spec = pl.BlockSpec((tm, d_out), lambda i: (i, 0))
    if emit_f8_adj:
        body = _layer_cast_body
        s_dtype = jnp.bfloat16
        out_shape = [out_shape, jax.ShapeDtypeStruct((N, N), _F8)]
        out_spec = [out_spec, pl.BlockSpec((tm, N), lambda i: (i, 0))]
    else:
        body = _layer_body
        s_dtype = adj.dtype
    return pl.pallas_call(
        body,
        grid=(N // tm,),
        in_specs=[
            pl.BlockSpec((N, d_in), lambda i: (0, 0)),
            pl.BlockSpec((d_in, d_out), lambda i: (0, 0)),
            pl.BlockSpec((tm, N), lambda i: (i, 0)),
        ],
        out_specs=out_spec,
        out_shape=out_shape,
        scratch_shapes=[pltpu.VMEM((N, d_out), s_dtype)],
    )(features, W, adj)


def _final_body(zh_blk_ref, zh_full_ref, out_ref, zt_ref):
    @pl.when(pl.program_id(0) == 0)
    def _():
        zt_ref[...] = zh_full_ref[...].T.astype(jnp.bfloat16)

    lhs = zh_blk_ref[...].astype(jnp.bfloat16)
    acc = jnp.dot(lhs, zt_ref[...], preferred_element_type=jnp.float32)
    out_ref[...] = 0.5 * jnp.tanh(0.5 * acc) + 0.5


def _gram_sigmoid(z_hat, tm):
    N, d = z_hat.shape
    return pl.pallas_call(
        _final_body,
        grid=(N // tm,),
        in_specs=[
            pl.BlockSpec((tm, d), lambda i: (i, 0)),
            pl.BlockSpec((N, d), lambda i: (0, 0)),
        ],
        out_specs=pl.BlockSpec((tm, N), lambda i: (i, 0)),
        out_shape=jax.ShapeDtypeStruct((N, N), jnp.float32),
        scratch_shapes=[pltpu.VMEM((d, N), jnp.bfloat16)],
    )(z_hat, z_hat)


def _pick_tm(N, pref):
    for tm in (pref, 400, 200, 100):
        if tm <= N and N % tm == 0:
            return tm
    return N


def kernel(z_igae, adj, W4, W5, W6):
    N = adj.shape[0]
    z1, adj_f8 = _gcn_layer(z_igae, W4, adj, _pick_tm(N, 400),
                            emit_f8_adj=True)
    z2 = _gcn_layer(z1, W5, adj_f8, _pick_tm(N, 1000))
    z_hat = _gcn_layer(z2, W6, adj_f8, _pick_tm(N, 1000))
    z_hat_adj = _gram_sigmoid(z_hat, _pick_tm(N, 400))
    return (z_hat, z_hat_adj)


# speedup vs baseline: 1.3871x; 1.0091x over previous
import jax
import jax.numpy as jnp
from jax.experimental import pallas as pl
from jax.experimental.pallas import tpu as pltpu

_TM = 200


_F8 = jnp.float8_e4m3fn


def _support(f_ref, w_ref, s_ref):
    @pl.when(pl.program_id(0) == 0)
    def _():
        s = jnp.dot(f_ref[...].astype(jnp.bfloat16),
                    w_ref[...].astype(jnp.bfloat16),
                    preferred_element_type=jnp.float32)
        s_ref[...] = jnp.tanh(s).astype(s_ref.dtype)


def _layer_cast_body(f_ref, w_ref, adj_ref, out_ref, adj_f8_ref, s_ref):
    _support(f_ref, w_ref, s_ref)
    a32 = adj_ref[...]
    adj_f8_ref[...] = a32.astype(_F8)
    out_ref[...] = jnp.dot(a32.astype(jnp.bfloat16), s_ref[...],
                           preferred_element_type=jnp.float32
                           ).astype(out_ref.dtype)


def _layer_body(f_ref, w_ref, adj_ref, out_ref, s_ref):
    _support(f_ref, w_ref, s_ref)
    out_ref[...] = jnp.dot(adj_ref[...], s_ref[...],
                           preferred_element_type=jnp.float32
                           ).astype(out_ref.dtype)


def _gcn_layer(features, W, adj, tm, emit_f8_adj=False, out_dtype=jnp.float32):
    N, d_in = features.shape
    d_out = W.shape[1]
    out_shape = jax.ShapeDtypeStruct((N, d_out), out_dtype)
    out_spec = pl.BlockSpec((tm, d_out), lambda i: (i, 0))
    if emit_f8_adj:
        body = _layer_cast_body
        s_dtype = jnp.bfloat16
        out_shape = [out_shape, jax.ShapeDtypeStruct((N, N), _F8)]
        out_spec = [out_spec, pl.BlockSpec((tm, N), lambda i: (i, 0))]
    else:
        body = _layer_body
        s_dtype = adj.dtype
    return pl.pallas_call(
        body,
        grid=(N // tm,),
        in_specs=[
            pl.BlockSpec((N, d_in), lambda i: (0, 0)),
            pl.BlockSpec((d_in, d_out), lambda i: (0, 0)),
            pl.BlockSpec((tm, N), lambda i: (i, 0)),
        ],
        out_specs=out_spec,
        out_shape=out_shape,
        scratch_shapes=[pltpu.VMEM((N, d_out), s_dtype)],
    )(features, W, adj)


def _final_body(zh_blk_ref, zh_full_ref, out_ref, zt_ref):
    @pl.when(pl.program_id(0) == 0)
    def _():
        zt_ref[...] = zh_full_ref[...].T.astype(jnp.bfloat16)

    lhs = zh_blk_ref[...].astype(jnp.bfloat16)
    acc = jnp.dot(lhs, zt_ref[...], preferred_element_type=jnp.float32)
    out_ref[...] = 0.5 * jnp.tanh(0.5 * acc) + 0.5


def _gram_sigmoid(z_hat, tm):
    N, d = z_hat.shape
    return pl.pallas_call(
        _final_body,
        grid=(N // tm,),
        in_specs=[
            pl.BlockSpec((tm, d), lambda i: (i, 0)),
            pl.BlockSpec((N, d), lambda i: (0, 0)),
        ],
        out_specs=pl.BlockSpec((tm, N), lambda i: (i, 0)),
        out_shape=jax.ShapeDtypeStruct((N, N), jnp.float32),
        scratch_shapes=[pltpu.VMEM((d, N), jnp.bfloat16)],
    )(z_hat, z_hat)


def _pick_tm(N, pref):
    for tm in (pref, 400, 200, 100):
        if tm <= N and N % tm == 0:
            return tm
    return N


def kernel(z_igae, adj, W4, W5, W6):
    N = adj.shape[0]
    z1, adj_f8 = _gcn_layer(z_igae, W4, adj, _pick_tm(N, 400),
                            emit_f8_adj=True, out_dtype=jnp.bfloat16)
    z2 = _gcn_layer(z1, W5, adj_f8, _pick_tm(N, 1000),
                    out_dtype=jnp.bfloat16)
    z_hat = _gcn_layer(z2, W6, adj_f8, _pick_tm(N, 1000))
    z_hat_adj = _gram_sigmoid(z_hat, _pick_tm(N, 400))
    return (z_hat, z_hat_adj)
